# Initial kernel scaffold; baseline (speedup 1.0000x reference)
import jax
import jax.numpy as jnp
from jax.experimental import pallas as pl


def kernel(coords, normals, curvatures, edge_index, W1, b1, W2, b2, W3, b3, Wp, bp, gamma_e, beta_e, gamma_n, beta_n):
    raise NotImplementedError("write your pallas kernel here")


# speedup vs baseline: 7.4467x; 7.4467x over previous
import functools

import jax
import jax.numpy as jnp
from jax import lax
from jax.experimental import pallas as pl
from jax.experimental.pallas import tpu as pltpu
from jax.experimental.pallas import tpu_sc as plsc

_F32 = jnp.float32
_NC = 2
_NS = 16
_NW = _NC * _NS
_BS = 125
_KB = 8
_CH = _BS * _KB
_BLK = 2000
_BLKN = 2000


def _gelu(x):
    return 0.5 * x * (1.0 + lax.erf(x * 0.7071067811865476))


def _mesh():
    return plsc.VectorSubcoreMesh(core_axis_name="c", subcore_axis_name="s")


def _stage1_gather(node_tab, row2d, col2d):
    E = row2d.shape[0] * _BS
    EPW = E // _NW
    NCHUNK = EPW // _CH
    RPW = EPW // _BS

    @functools.partial(
        pl.kernel, mesh=_mesh(),
        out_type=(jax.ShapeDtypeStruct((E, 8), _F32),
                  jax.ShapeDtypeStruct((E, 8), _F32)),
        scratch_types=[
            pltpu.VMEM((_KB, _BS), jnp.int32),
            pltpu.VMEM((_KB, _BS), jnp.int32),
            pltpu.VMEM((_CH, 8), _F32),
            pltpu.VMEM((_CH, 8), _F32),
            pltpu.SemaphoreType.DMA,
        ],
        compiler_params=pltpu.CompilerParams(use_tc_tiling_on_sc=False),
    )
    def sc_gather(tab_hbm, row_hbm, col_hbm, rout_hbm, cout_hbm,
                  ridx_v, cidx_v, rbuf_v, cbuf_v, sem):
        cid = lax.axis_index("c")
        sid = lax.axis_index("s")
        wid = cid * _NS + sid

        def body(ci, carry):
            r0 = wid * RPW + ci * _KB
            eoff = wid * EPW + ci * _CH
            pltpu.sync_copy(row_hbm.at[pl.ds(r0, _KB)], ridx_v)
            pltpu.sync_copy(col_hbm.at[pl.ds(r0, _KB)], cidx_v)
            copies = []
            for j in range(_KB):
                copies.append(pltpu.async_copy(
                    tab_hbm.at[ridx_v.at[j]],
                    rbuf_v.at[pl.ds(j * _BS, _BS)], sem))
                copies.append(pltpu.async_copy(
                    tab_hbm.at[cidx_v.at[j]],
                    cbuf_v.at[pl.ds(j * _BS, _BS)], sem))
            for cp in copies:
                cp.wait()
            pltpu.sync_copy(rbuf_v, rout_hbm.at[pl.ds(eoff, _CH)])
            pltpu.sync_copy(cbuf_v, cout_hbm.at[pl.ds(eoff, _CH)])
            return carry

        lax.fori_loop(0, NCHUNK, body, 0)

    return sc_gather(node_tab, row2d, col2d)


def _stage2_mlp(row_feat, col_feat, W1, b1, W2, b2, W3, b3):
    E = row_feat.shape[0]
    eps = 1e-8

    def mlp_body(rf, cf, w1r, b1r, w2r, b2r, w3r, b3r, h_ref, st_ref):
        r = rf[...]
        c = cf[...]
        delta = c[:, 0:3] - r[:, 0:3]
        rn = r[:, 3:6]
        cn = c[:, 3:6]
        ndot = jnp.sum(rn * cn, axis=1, keepdims=True)
        dnorm = jnp.sqrt(jnp.sum(delta * delta, axis=1, keepdims=True)) + eps
        cr = jnp.clip(jnp.sum(rn * delta, axis=1, keepdims=True) / dnorm,
                      -1 + eps, 1 - eps)
        cc = jnp.clip(jnp.sum(cn * delta, axis=1, keepdims=True) / dnorm,
                      -1 + eps, 1 - eps)
        cd = c[:, 6:8] - r[:, 6:8]
        f = jnp.concatenate([delta, ndot, cr, cc, cd], axis=1)
        h = _gelu(jnp.dot(f, w1r[...], preferred_element_type=_F32,
                          precision=lax.Precision.HIGHEST) + b1r[...])
        h = _gelu(jnp.dot(h, w2r[...], preferred_element_type=_F32,
                          precision=lax.Precision.HIGHEST) + b2r[...])
        h = jnp.dot(h, w3r[...], preferred_element_type=_F32,
                    precision=lax.Precision.HIGHEST) + b3r[...]

        @pl.when(pl.program_id(0) == 0)
        def _init():
            st_ref[...] = jnp.zeros_like(st_ref)

        st_ref[0:1, 0:16] += jnp.sum(h, axis=0)[None, :]
        st_ref[1:2, 0:16] += jnp.sum(h * h, axis=0)[None, :]
        h_ref[0] = h[:, 0:8]
        h_ref[1] = h[:, 8:16]

    return pl.pallas_call(
        mlp_body,
        grid=(E // _BLK,),
        in_specs=[
            pl.BlockSpec((_BLK, 8), lambda i: (i, 0)),
            pl.BlockSpec((_BLK, 8), lambda i: (i, 0)),
            pl.BlockSpec((8, 64), lambda i: (0, 0)),
            pl.BlockSpec((1, 64), lambda i: (0, 0)),
            pl.BlockSpec((64, 32), lambda i: (0, 0)),
            pl.BlockSpec((1, 32), lambda i: (0, 0)),
            pl.BlockSpec((32, 16), lambda i: (0, 0)),
            pl.BlockSpec((1, 16), lambda i: (0, 0)),
        ],
        out_specs=[
            pl.BlockSpec((2, _BLK, 8), lambda i: (0, i, 0)),
            pl.BlockSpec((8, 128), lambda i: (0, 0)),
        ],
        out_shape=[
            jax.ShapeDtypeStruct((2, E, 8), _F32),
            jax.ShapeDtypeStruct((8, 128), _F32),
        ],
    )(row_feat, col_feat, W1.T, b1.reshape(1, 64), W2.T, b2.reshape(1, 32),
      W3.T, b3.reshape(1, 16))


def _stage3_scatter(h2, row2d, col2d, Nn):
    E = row2d.shape[0] * _BS
    EPT = E // _NS
    NCHUNK2 = EPT // _CH
    RPT = EPT // _BS
    zh = jnp.zeros((Nn, 8), _F32)
    zd = jnp.zeros((Nn,), _F32)
    ones_bs = jnp.ones((_BS,), _F32)

    @functools.partial(
        pl.kernel, mesh=_mesh(),
        out_type=(jax.ShapeDtypeStruct((_NC, Nn, 8), _F32),
                  jax.ShapeDtypeStruct((_NC, Nn), _F32)),
        scratch_types=[
            pltpu.VMEM((_KB, _BS), jnp.int32),
            pltpu.VMEM((_KB, _BS), jnp.int32),
            pltpu.VMEM((_CH, 8), _F32),
            pltpu.VMEM((_BS,), _F32),
            pltpu.VMEM_SHARED((Nn, 8), _F32),
            pltpu.VMEM_SHARED((Nn,), _F32),
            pltpu.SemaphoreType.DMA,
        ],
        compiler_params=pltpu.CompilerParams(use_tc_tiling_on_sc=False),
    )
    def sc_scatter(h2_hbm, row_hbm, col_hbm, zh_hbm, zd_hbm, ones_hbm,
                   outh_hbm, outd_hbm,
                   ridx_v, cidx_v, upd_v, ones_v, acc_sh, deg_sh, sem):
        cid = lax.axis_index("c")
        sid = lax.axis_index("s")

        @pl.when(sid == 0)
        def _zero():
            pltpu.sync_copy(zh_hbm, acc_sh)
            pltpu.sync_copy(zd_hbm, deg_sh)

        pltpu.sync_copy(ones_hbm, ones_v)
        plsc.subcore_barrier()

        def body(ci, carry):
            r0 = sid * RPT + ci * _KB
            eoff = sid * EPT + ci * _CH
            pltpu.sync_copy(h2_hbm.at[cid, pl.ds(eoff, _CH)], upd_v)
            pltpu.sync_copy(row_hbm.at[pl.ds(r0, _KB)], ridx_v)
            pltpu.sync_copy(col_hbm.at[pl.ds(r0, _KB)], cidx_v)
            for j in range(_KB):
                u = upd_v.at[pl.ds(j * _BS, _BS)]
                pltpu.sync_copy(u, acc_sh.at[ridx_v.at[j]], add=True)
                pltpu.sync_copy(u, acc_sh.at[cidx_v.at[j]], add=True)

                @pl.when(cid == 0)
                def _d0():
                    pltpu.sync_copy(ones_v, deg_sh.at[ridx_v.at[j]],
                                    add=True)

                @pl.when(cid == 1)
                def _d1():
                    pltpu.sync_copy(ones_v, deg_sh.at[cidx_v.at[j]],
                                    add=True)

            return carry

        lax.fori_loop(0, NCHUNK2, body, 0)
        plsc.subcore_barrier()

        @pl.when(sid == 0)
        def _out():
            pltpu.sync_copy(acc_sh, outh_hbm.at[cid])
            pltpu.sync_copy(deg_sh, outd_hbm.at[cid])

    return sc_scatter(h2, row2d, col2d, zh, zd, ones_bs)


def _stage4_node(acc2, deg2, estats, E, Wp, bp, gamma_e, beta_e,
                 gamma_n, beta_n):
    Nn = acc2.shape[1]
    ssum = estats[0, :16]
    ssq = estats[1, :16]
    mu_e = ssum / E
    var_e = ssq / E - mu_e * mu_e
    sig_e = jnp.sqrt(var_e + 1e-5)
    scale_e = (gamma_e / sig_e).reshape(1, 16)
    shift_e = (beta_e - gamma_e * mu_e / sig_e).reshape(1, 16)

    def node_body(ah, ad, wpr, bpr, scr, shr, y_ref, st_ref):
        s = jnp.concatenate([ah[0], ah[1]], axis=1)
        d = ad[:, 0:1] + ad[:, 1:2]
        pre = (s * scr[...] + d * shr[...]) / jnp.maximum(d, 1.0)
        y = jnp.dot(pre, wpr[...], preferred_element_type=_F32,
                    precision=lax.Precision.HIGHEST) + bpr[...]

        @pl.when(pl.program_id(0) == 0)
        def _init():
            st_ref[...] = jnp.zeros_like(st_ref)

        st_ref[0:1, 0:16] += jnp.sum(y, axis=0)[None, :]
        st_ref[1:2, 0:16] += jnp.sum(y * y, axis=0)[None, :]
        y_ref[...] = y

    y_raw, nstats = pl.pallas_call(
        node_body,
        grid=(Nn // _BLKN,),
        in_specs=[
            pl.BlockSpec((2, _BLKN, 8), lambda i: (0, i, 0)),
            pl.BlockSpec((_BLKN, 2), lambda i: (i, 0)),
            pl.BlockSpec((16, 16), lambda i: (0, 0)),
            pl.BlockSpec((1, 16), lambda i: (0, 0)),
            pl.BlockSpec((1, 16), lambda i: (0, 0)),
            pl.BlockSpec((1, 16), lambda i: (0, 0)),
        ],
        out_specs=[
            pl.BlockSpec((_BLKN, 16), lambda i: (i, 0)),
            pl.BlockSpec((8, 128), lambda i: (0, 0)),
        ],
        out_shape=[
            jax.ShapeDtypeStruct((Nn, 16), _F32),
            jax.ShapeDtypeStruct((8, 128), _F32),
        ],
    )(acc2, deg2, Wp.T, bp.reshape(1, 16), scale_e, shift_e)

    mu_n = nstats[0, :16] / Nn
    var_n = nstats[1, :16] / Nn - (nstats[0, :16] / Nn) ** 2
    sig_n = jnp.sqrt(var_n + 1e-5)
    inv_n = (gamma_n / sig_n).reshape(1, 16)
    off_n = (beta_n - mu_n * gamma_n / sig_n).reshape(1, 16)

    def norm_body(yr, ir, orr, o_ref):
        o_ref[...] = yr[...] * ir[...] + orr[...]

    return pl.pallas_call(
        norm_body,
        grid=(Nn // _BLKN,),
        in_specs=[
            pl.BlockSpec((_BLKN, 16), lambda i: (i, 0)),
            pl.BlockSpec((1, 16), lambda i: (0, 0)),
            pl.BlockSpec((1, 16), lambda i: (0, 0)),
        ],
        out_specs=pl.BlockSpec((_BLKN, 16), lambda i: (i, 0)),
        out_shape=jax.ShapeDtypeStruct((Nn, 16), _F32),
    )(y_raw, inv_n, off_n)


def kernel(coords, normals, curvatures, edge_index, W1, b1, W2, b2, W3, b3,
           Wp, bp, gamma_e, beta_e, gamma_n, beta_n):
    Nn = coords.shape[0]
    E = edge_index.shape[1]
    node_tab = jnp.concatenate([coords, normals, curvatures[:, :2]], axis=1)
    row2d = edge_index[0].reshape(E // _BS, _BS)
    col2d = edge_index[1].reshape(E // _BS, _BS)

    row_feat, col_feat = _stage1_gather(node_tab, row2d, col2d)
    h2, estats = _stage2_mlp(row_feat, col_feat, W1, b1, W2, b2, W3, b3)
    acc2, deg2 = _stage3_scatter(h2, row2d, col2d, Nn)
    return _stage4_node(acc2, deg2.T, estats, E, Wp, bp, gamma_e, beta_e,
                        gamma_n, beta_n)


# speedup vs baseline: 23.0607x; 3.0968x over previous
import functools

import jax
import jax.numpy as jnp
from jax import lax
from jax.experimental import pallas as pl
from jax.experimental.pallas import tpu as pltpu
from jax.experimental.pallas import tpu_sc as plsc

_F32 = jnp.float32
_NC = 2
_NS = 16
_NW = _NC * _NS
_BS = 125
_KB = 8
_CH = _BS * _KB
_BLK = 4000
_BLKN = 2000


def _gelu(x):
    return 0.5 * x * (1.0 + lax.erf(x * 0.7071067811865476))


def _mesh():
    return plsc.VectorSubcoreMesh(core_axis_name="c", subcore_axis_name="s")


def _stage1_gather(node_tab, row2d, col2d):
    E = row2d.shape[0] * _BS
    EPW = E // _NW
    NCHUNK = EPW // _CH
    RPW = EPW // _BS

    @functools.partial(
        pl.kernel, mesh=_mesh(),
        out_type=(jax.ShapeDtypeStruct((E, 8), _F32),
                  jax.ShapeDtypeStruct((E, 8), _F32)),
        scratch_types=[
            pltpu.VMEM((_KB, _BS), jnp.int32),
            pltpu.VMEM((_KB, _BS), jnp.int32),
            pltpu.VMEM((_CH, 8), _F32),
            pltpu.VMEM((_CH, 8), _F32),
            pltpu.SemaphoreType.DMA,
        ],
        compiler_params=pltpu.CompilerParams(use_tc_tiling_on_sc=False),
    )
    def sc_gather(tab_hbm, row_hbm, col_hbm, rout_hbm, cout_hbm,
                  ridx_v, cidx_v, rbuf_v, cbuf_v, sem):
        cid = lax.axis_index("c")
        sid = lax.axis_index("s")
        wid = cid * _NS + sid

        def body(ci, carry):
            r0 = wid * RPW + ci * _KB
            eoff = wid * EPW + ci * _CH
            pltpu.sync_copy(row_hbm.at[pl.ds(r0, _KB)], ridx_v)
            pltpu.sync_copy(col_hbm.at[pl.ds(r0, _KB)], cidx_v)
            copies = []
            for j in range(_KB):
                copies.append(pltpu.async_copy(
                    tab_hbm.at[ridx_v.at[j]],
                    rbuf_v.at[pl.ds(j * _BS, _BS)], sem))
                copies.append(pltpu.async_copy(
                    tab_hbm.at[cidx_v.at[j]],
                    cbuf_v.at[pl.ds(j * _BS, _BS)], sem))
            for cp in copies:
                cp.wait()
            pltpu.sync_copy(rbuf_v, rout_hbm.at[pl.ds(eoff, _CH)])
            pltpu.sync_copy(cbuf_v, cout_hbm.at[pl.ds(eoff, _CH)])
            return carry

        lax.fori_loop(0, NCHUNK, body, 0)

    return sc_gather(node_tab, row2d, col2d)


def _stage2_mlp(row_feat, col_feat, W1, b1, W2, b2, W3, b3):
    E = row_feat.shape[0]
    eps = 1e-8

    def mlp_body(rf, cf, w1r, b1r, w2r, b2r, w3r, b3r, h_ref, st_ref):
        r = rf[...].T
        c = cf[...].T
        delta = c[0:3] - r[0:3]
        rn = r[3:6]
        cn = c[3:6]
        ndot = jnp.sum(rn * cn, axis=0, keepdims=True)
        dnorm = jnp.sqrt(jnp.sum(delta * delta, axis=0, keepdims=True)) + eps
        cr = jnp.clip(jnp.sum(rn * delta, axis=0, keepdims=True) / dnorm,
                      -1 + eps, 1 - eps)
        cc = jnp.clip(jnp.sum(cn * delta, axis=0, keepdims=True) / dnorm,
                      -1 + eps, 1 - eps)
        cd = c[6:8] - r[6:8]
        f = jnp.concatenate([delta, ndot, cr, cc, cd], axis=0)
        h = _gelu(jnp.dot(w1r[...], f, preferred_element_type=_F32,
                          precision=lax.Precision.HIGHEST) + b1r[...])
        h = _gelu(jnp.dot(w2r[...], h, preferred_element_type=_F32,
                          precision=lax.Precision.HIGHEST) + b2r[...])
        h = jnp.dot(w3r[...], h, preferred_element_type=_F32,
                    precision=lax.Precision.HIGHEST) + b3r[...]

        @pl.when(pl.program_id(0) == 0)
        def _init():
            st_ref[...] = jnp.zeros_like(st_ref)

        st_ref[0:16, 0:1] += jnp.sum(h, axis=1, keepdims=True)
        st_ref[0:16, 1:2] += jnp.sum(h * h, axis=1, keepdims=True)
        ht = h.T
        h_ref[0] = ht[:, 0:8]
        h_ref[1] = ht[:, 8:16]

    return pl.pallas_call(
        mlp_body,
        grid=(E // _BLK,),
        in_specs=[
            pl.BlockSpec((_BLK, 8), lambda i: (i, 0)),
            pl.BlockSpec((_BLK, 8), lambda i: (i, 0)),
            pl.BlockSpec((64, 8), lambda i: (0, 0)),
            pl.BlockSpec((64, 1), lambda i: (0, 0)),
            pl.BlockSpec((32, 64), lambda i: (0, 0)),
            pl.BlockSpec((32, 1), lambda i: (0, 0)),
            pl.BlockSpec((16, 32), lambda i: (0, 0)),
            pl.BlockSpec((16, 1), lambda i: (0, 0)),
        ],
        out_specs=[
            pl.BlockSpec((2, _BLK, 8), lambda i: (0, i, 0)),
            pl.BlockSpec((16, 128), lambda i: (0, 0)),
        ],
        out_shape=[
            jax.ShapeDtypeStruct((2, E, 8), _F32),
            jax.ShapeDtypeStruct((16, 128), _F32),
        ],
    )(row_feat, col_feat, W1, b1.reshape(64, 1), W2, b2.reshape(32, 1),
      W3, b3.reshape(16, 1))


def _stage3_scatter(h2, row2d, col2d, Nn):
    E = row2d.shape[0] * _BS
    EPT = E // _NS
    NCHUNK2 = EPT // _CH
    RPT = EPT // _BS
    zh = jnp.zeros((Nn, 8), _F32)
    zd = jnp.zeros((Nn,), _F32)
    ones_bs = jnp.ones((_BS,), _F32)

    @functools.partial(
        pl.kernel, mesh=_mesh(),
        out_type=(jax.ShapeDtypeStruct((_NC, Nn, 8), _F32),
                  jax.ShapeDtypeStruct((_NC, Nn), _F32)),
        scratch_types=[
            pltpu.VMEM((_KB, _BS), jnp.int32),
            pltpu.VMEM((_KB, _BS), jnp.int32),
            pltpu.VMEM((_CH, 8), _F32),
            pltpu.VMEM((_BS,), _F32),
            pltpu.VMEM_SHARED((Nn, 8), _F32),
            pltpu.VMEM_SHARED((Nn,), _F32),
            pltpu.SemaphoreType.DMA,
        ],
        compiler_params=pltpu.CompilerParams(use_tc_tiling_on_sc=False),
    )
    def sc_scatter(h2_hbm, row_hbm, col_hbm, zh_hbm, zd_hbm, ones_hbm,
                   outh_hbm, outd_hbm,
                   ridx_v, cidx_v, upd_v, ones_v, acc_sh, deg_sh, sem):
        cid = lax.axis_index("c")
        sid = lax.axis_index("s")

        @pl.when(sid == 0)
        def _zero():
            pltpu.sync_copy(zh_hbm, acc_sh)
            pltpu.sync_copy(zd_hbm, deg_sh)

        pltpu.sync_copy(ones_hbm, ones_v)
        plsc.subcore_barrier()

        def body(ci, carry):
            r0 = sid * RPT + ci * _KB
            eoff = sid * EPT + ci * _CH
            pltpu.sync_copy(h2_hbm.at[cid, pl.ds(eoff, _CH)], upd_v)
            pltpu.sync_copy(row_hbm.at[pl.ds(r0, _KB)], ridx_v)
            pltpu.sync_copy(col_hbm.at[pl.ds(r0, _KB)], cidx_v)
            for j in range(_KB):
                u = upd_v.at[pl.ds(j * _BS, _BS)]
                pltpu.sync_copy(u, acc_sh.at[ridx_v.at[j]], add=True)
                pltpu.sync_copy(u, acc_sh.at[cidx_v.at[j]], add=True)

                @pl.when(cid == 0)
                def _d0():
                    pltpu.sync_copy(ones_v, deg_sh.at[ridx_v.at[j]],
                                    add=True)

                @pl.when(cid == 1)
                def _d1():
                    pltpu.sync_copy(ones_v, deg_sh.at[cidx_v.at[j]],
                                    add=True)

            return carry

        lax.fori_loop(0, NCHUNK2, body, 0)
        plsc.subcore_barrier()

        @pl.when(sid == 0)
        def _out():
            pltpu.sync_copy(acc_sh, outh_hbm.at[cid])
            pltpu.sync_copy(deg_sh, outd_hbm.at[cid])

    return sc_scatter(h2, row2d, col2d, zh, zd, ones_bs)


def _stage4_node(acc2, deg2, estats, E, Wp, bp, gamma_e, beta_e,
                 gamma_n, beta_n):
    Nn = acc2.shape[1]
    ssum = estats[:16, 0]
    ssq = estats[:16, 1]
    mu_e = ssum / E
    var_e = ssq / E - mu_e * mu_e
    sig_e = jnp.sqrt(var_e + 1e-5)
    scale_e = (gamma_e / sig_e).reshape(1, 16)
    shift_e = (beta_e - gamma_e * mu_e / sig_e).reshape(1, 16)

    def node_body(ah, ad, wpr, bpr, scr, shr, y_ref, st_ref):
        s = jnp.concatenate([ah[0], ah[1]], axis=1)
        d = ad[:, 0:1] + ad[:, 1:2]
        pre = (s * scr[...] + d * shr[...]) / jnp.maximum(d, 1.0)
        y = jnp.dot(pre, wpr[...], preferred_element_type=_F32,
                    precision=lax.Precision.HIGHEST) + bpr[...]

        @pl.when(pl.program_id(0) == 0)
        def _init():
            st_ref[...] = jnp.zeros_like(st_ref)

        st_ref[0:1, 0:16] += jnp.sum(y, axis=0)[None, :]
        st_ref[1:2, 0:16] += jnp.sum(y * y, axis=0)[None, :]
        y_ref[...] = y

    y_raw, nstats = pl.pallas_call(
        node_body,
        grid=(Nn // _BLKN,),
        in_specs=[
            pl.BlockSpec((2, _BLKN, 8), lambda i: (0, i, 0)),
            pl.BlockSpec((_BLKN, 2), lambda i: (i, 0)),
            pl.BlockSpec((16, 16), lambda i: (0, 0)),
            pl.BlockSpec((1, 16), lambda i: (0, 0)),
            pl.BlockSpec((1, 16), lambda i: (0, 0)),
            pl.BlockSpec((1, 16), lambda i: (0, 0)),
        ],
        out_specs=[
            pl.BlockSpec((_BLKN, 16), lambda i: (i, 0)),
            pl.BlockSpec((8, 128), lambda i: (0, 0)),
        ],
        out_shape=[
            jax.ShapeDtypeStruct((Nn, 16), _F32),
            jax.ShapeDtypeStruct((8, 128), _F32),
        ],
    )(acc2, deg2, Wp.T, bp.reshape(1, 16), scale_e, shift_e)

    mu_n = nstats[0, :16] / Nn
    var_n = nstats[1, :16] / Nn - (nstats[0, :16] / Nn) ** 2
    sig_n = jnp.sqrt(var_n + 1e-5)
    inv_n = (gamma_n / sig_n).reshape(1, 16)
    off_n = (beta_n - mu_n * gamma_n / sig_n).reshape(1, 16)

    def norm_body(yr, ir, orr, o_ref):
        o_ref[...] = yr[...] * ir[...] + orr[...]

    return pl.pallas_call(
        norm_body,
        grid=(Nn // _BLKN,),
        in_specs=[
            pl.BlockSpec((_BLKN, 16), lambda i: (i, 0)),
            pl.BlockSpec((1, 16), lambda i: (0, 0)),
            pl.BlockSpec((1, 16), lambda i: (0, 0)),
        ],
        out_specs=pl.BlockSpec((_BLKN, 16), lambda i: (i, 0)),
        out_shape=jax.ShapeDtypeStruct((Nn, 16), _F32),
    )(y_raw, inv_n, off_n)


def kernel(coords, normals, curvatures, edge_index, W1, b1, W2, b2, W3, b3,
           Wp, bp, gamma_e, beta_e, gamma_n, beta_n):
    Nn = coords.shape[0]
    E = edge_index.shape[1]
    node_tab = jnp.concatenate([coords, normals, curvatures[:, :2]], axis=1)
    row2d = edge_index[0].reshape(E // _BS, _BS)
    col2d = edge_index[1].reshape(E // _BS, _BS)

    row_feat, col_feat = _stage1_gather(node_tab, row2d, col2d)
    h2, estats = _stage2_mlp(row_feat, col_feat, W1, b1, W2, b2, W3, b3)
    acc2, deg2 = _stage3_scatter(h2, row2d, col2d, Nn)
    return _stage4_node(acc2, deg2.T, estats, E, Wp, bp, gamma_e, beta_e,
                        gamma_n, beta_n)


# speedup vs baseline: 23.9099x; 1.0368x over previous
import functools

import jax
import jax.numpy as jnp
from jax import lax
from jax.experimental import pallas as pl
from jax.experimental.pallas import tpu as pltpu
from jax.experimental.pallas import tpu_sc as plsc

_F32 = jnp.float32
_NC = 2
_NS = 16
_NW = _NC * _NS
_BS = 125
_KB = 8
_CH = _BS * _KB
_BLK = 16000
_BLKN = 2000


def _gelu(x):
    return 0.5 * x * (1.0 + lax.erf(x * 0.7071067811865476))


def _mesh():
    return plsc.VectorSubcoreMesh(core_axis_name="c", subcore_axis_name="s")


def _stage1_gather(node_tab, row2d, col2d):
    E = row2d.shape[0] * _BS
    EPW = E // _NW
    NCHUNK = EPW // _CH
    RPW = EPW // _BS

    @functools.partial(
        pl.kernel, mesh=_mesh(),
        out_type=(jax.ShapeDtypeStruct((E, 8), _F32),
                  jax.ShapeDtypeStruct((E, 8), _F32)),
        scratch_types=[
            pltpu.VMEM((_KB, _BS), jnp.int32),
            pltpu.VMEM((_KB, _BS), jnp.int32),
            pltpu.VMEM((_CH, 8), _F32),
            pltpu.VMEM((_CH, 8), _F32),
            pltpu.SemaphoreType.DMA,
        ],
        compiler_params=pltpu.CompilerParams(use_tc_tiling_on_sc=False),
    )
    def sc_gather(tab_hbm, row_hbm, col_hbm, rout_hbm, cout_hbm,
                  ridx_v, cidx_v, rbuf_v, cbuf_v, sem):
        cid = lax.axis_index("c")
        sid = lax.axis_index("s")
        wid = cid * _NS + sid

        def body(ci, carry):
            r0 = wid * RPW + ci * _KB
            eoff = wid * EPW + ci * _CH
            pltpu.sync_copy(row_hbm.at[pl.ds(r0, _KB)], ridx_v)
            pltpu.sync_copy(col_hbm.at[pl.ds(r0, _KB)], cidx_v)
            copies = []
            for j in range(_KB):
                copies.append(pltpu.async_copy(
                    tab_hbm.at[ridx_v.at[j]],
                    rbuf_v.at[pl.ds(j * _BS, _BS)], sem))
                copies.append(pltpu.async_copy(
                    tab_hbm.at[cidx_v.at[j]],
                    cbuf_v.at[pl.ds(j * _BS, _BS)], sem))
            for cp in copies:
                cp.wait()
            pltpu.sync_copy(rbuf_v, rout_hbm.at[pl.ds(eoff, _CH)])
            pltpu.sync_copy(cbuf_v, cout_hbm.at[pl.ds(eoff, _CH)])
            return carry

        lax.fori_loop(0, NCHUNK, body, 0)

    return sc_gather(node_tab, row2d, col2d)


def _stage2_mlp(row_feat, col_feat, W1, b1, W2, b2, W3, b3):
    E = row_feat.shape[1]
    eps = 1e-8

    def mlp_body(rf, cf, w1r, b1r, w2r, b2r, w3r, b3r, h_ref, st_ref):
        r = rf[...]
        c = cf[...]
        delta = c[0:3] - r[0:3]
        rn = r[3:6]
        cn = c[3:6]
        ndot = jnp.sum(rn * cn, axis=0, keepdims=True)
        dnorm = jnp.sqrt(jnp.sum(delta * delta, axis=0, keepdims=True)) + eps
        cr = jnp.clip(jnp.sum(rn * delta, axis=0, keepdims=True) / dnorm,
                      -1 + eps, 1 - eps)
        cc = jnp.clip(jnp.sum(cn * delta, axis=0, keepdims=True) / dnorm,
                      -1 + eps, 1 - eps)
        cd = c[6:8] - r[6:8]
        f = jnp.concatenate([delta, ndot, cr, cc, cd], axis=0)
        h = _gelu(jnp.dot(w1r[...], f, preferred_element_type=_F32,
                          precision=lax.Precision.HIGHEST) + b1r[...])
        h = _gelu(jnp.dot(w2r[...], h, preferred_element_type=_F32,
                          precision=lax.Precision.HIGHEST) + b2r[...])
        h = jnp.dot(w3r[...], h, preferred_element_type=_F32,
                    precision=lax.Precision.HIGHEST) + b3r[...]

        @pl.when(pl.program_id(0) == 0)
        def _init():
            st_ref[...] = jnp.zeros_like(st_ref)

        st_ref[0:16, 0:1] += jnp.sum(h, axis=1, keepdims=True)
        st_ref[0:16, 1:2] += jnp.sum(h * h, axis=1, keepdims=True)
        ht = h.T
        h_ref[0] = ht[:, 0:8]
        h_ref[1] = ht[:, 8:16]

    return pl.pallas_call(
        mlp_body,
        grid=(E // _BLK,),
        in_specs=[
            pl.BlockSpec((8, _BLK), lambda i: (0, i)),
            pl.BlockSpec((8, _BLK), lambda i: (0, i)),
            pl.BlockSpec((64, 8), lambda i: (0, 0)),
            pl.BlockSpec((64, 1), lambda i: (0, 0)),
            pl.BlockSpec((32, 64), lambda i: (0, 0)),
            pl.BlockSpec((32, 1), lambda i: (0, 0)),
            pl.BlockSpec((16, 32), lambda i: (0, 0)),
            pl.BlockSpec((16, 1), lambda i: (0, 0)),
        ],
        out_specs=[
            pl.BlockSpec((2, _BLK, 8), lambda i: (0, i, 0)),
            pl.BlockSpec((16, 128), lambda i: (0, 0)),
        ],
        out_shape=[
            jax.ShapeDtypeStruct((2, E, 8), _F32),
            jax.ShapeDtypeStruct((16, 128), _F32),
        ],
    )(row_feat, col_feat, W1, b1.reshape(64, 1), W2, b2.reshape(32, 1),
      W3, b3.reshape(16, 1))


def _stage3_scatter(h2, row2d, col2d, Nn):
    E = row2d.shape[0] * _BS
    EPT = E // _NS
    NCHUNK2 = EPT // _CH
    RPT = EPT // _BS
    zh = jnp.zeros((Nn, 8), _F32)
    zd = jnp.zeros((Nn,), _F32)
    ones_bs = jnp.ones((_BS,), _F32)

    @functools.partial(
        pl.kernel, mesh=_mesh(),
        out_type=(jax.ShapeDtypeStruct((_NC, Nn, 8), _F32),
                  jax.ShapeDtypeStruct((_NC, Nn), _F32)),
        scratch_types=[
            pltpu.VMEM((_KB, _BS), jnp.int32),
            pltpu.VMEM((_KB, _BS), jnp.int32),
            pltpu.VMEM((_CH, 8), _F32),
            pltpu.VMEM((_BS,), _F32),
            pltpu.VMEM_SHARED((Nn, 8), _F32),
            pltpu.VMEM_SHARED((Nn,), _F32),
            pltpu.SemaphoreType.DMA,
        ],
        compiler_params=pltpu.CompilerParams(use_tc_tiling_on_sc=False),
    )
    def sc_scatter(h2_hbm, row_hbm, col_hbm, zh_hbm, zd_hbm, ones_hbm,
                   outh_hbm, outd_hbm,
                   ridx_v, cidx_v, upd_v, ones_v, acc_sh, deg_sh, sem):
        cid = lax.axis_index("c")
        sid = lax.axis_index("s")

        @pl.when(sid == 0)
        def _zero():
            pltpu.sync_copy(zh_hbm, acc_sh)
            pltpu.sync_copy(zd_hbm, deg_sh)

        pltpu.sync_copy(ones_hbm, ones_v)
        plsc.subcore_barrier()

        def body(ci, carry):
            r0 = sid * RPT + ci * _KB
            eoff = sid * EPT + ci * _CH
            pltpu.sync_copy(h2_hbm.at[cid, pl.ds(eoff, _CH)], upd_v)
            pltpu.sync_copy(row_hbm.at[pl.ds(r0, _KB)], ridx_v)
            pltpu.sync_copy(col_hbm.at[pl.ds(r0, _KB)], cidx_v)
            for j in range(_KB):
                u = upd_v.at[pl.ds(j * _BS, _BS)]
                pltpu.sync_copy(u, acc_sh.at[ridx_v.at[j]], add=True)
                pltpu.sync_copy(u, acc_sh.at[cidx_v.at[j]], add=True)

                @pl.when(cid == 0)
                def _d0():
                    pltpu.sync_copy(ones_v, deg_sh.at[ridx_v.at[j]],
                                    add=True)

                @pl.when(cid == 1)
                def _d1():
                    pltpu.sync_copy(ones_v, deg_sh.at[cidx_v.at[j]],
                                    add=True)

            return carry

        lax.fori_loop(0, NCHUNK2, body, 0)
        plsc.subcore_barrier()

        @pl.when(sid == 0)
        def _out():
            pltpu.sync_copy(acc_sh, outh_hbm.at[cid])
            pltpu.sync_copy(deg_sh, outd_hbm.at[cid])

    return sc_scatter(h2, row2d, col2d, zh, zd, ones_bs)


def _stage4_node(acc2, deg2, estats, E, Wp, bp, gamma_e, beta_e,
                 gamma_n, beta_n):
    Nn = acc2.shape[1]
    ssum = estats[:16, 0]
    ssq = estats[:16, 1]
    mu_e = ssum / E
    var_e = ssq / E - mu_e * mu_e
    sig_e = jnp.sqrt(var_e + 1e-5)
    scale_e = (gamma_e / sig_e).reshape(1, 16)
    shift_e = (beta_e - gamma_e * mu_e / sig_e).reshape(1, 16)

    def node_body(ah, ad, wpr, bpr, scr, shr, y_ref, st_ref):
        s = jnp.concatenate([ah[0], ah[1]], axis=1)
        d = ad[:, 0:1] + ad[:, 1:2]
        pre = (s * scr[...] + d * shr[...]) / jnp.maximum(d, 1.0)
        y = jnp.dot(pre, wpr[...], preferred_element_type=_F32,
                    precision=lax.Precision.HIGHEST) + bpr[...]

        @pl.when(pl.program_id(0) == 0)
        def _init():
            st_ref[...] = jnp.zeros_like(st_ref)

        st_ref[0:1, 0:16] += jnp.sum(y, axis=0)[None, :]
        st_ref[1:2, 0:16] += jnp.sum(y * y, axis=0)[None, :]
        y_ref[...] = y

    y_raw, nstats = pl.pallas_call(
        node_body,
        grid=(Nn // _BLKN,),
        in_specs=[
            pl.BlockSpec((2, _BLKN, 8), lambda i: (0, i, 0)),
            pl.BlockSpec((_BLKN, 2), lambda i: (i, 0)),
            pl.BlockSpec((16, 16), lambda i: (0, 0)),
            pl.BlockSpec((1, 16), lambda i: (0, 0)),
            pl.BlockSpec((1, 16), lambda i: (0, 0)),
            pl.BlockSpec((1, 16), lambda i: (0, 0)),
        ],
        out_specs=[
            pl.BlockSpec((_BLKN, 16), lambda i: (i, 0)),
            pl.BlockSpec((8, 128), lambda i: (0, 0)),
        ],
        out_shape=[
            jax.ShapeDtypeStruct((Nn, 16), _F32),
            jax.ShapeDtypeStruct((8, 128), _F32),
        ],
    )(acc2, deg2, Wp.T, bp.reshape(1, 16), scale_e, shift_e)

    mu_n = nstats[0, :16] / Nn
    var_n = nstats[1, :16] / Nn - (nstats[0, :16] / Nn) ** 2
    sig_n = jnp.sqrt(var_n + 1e-5)
    inv_n = (gamma_n / sig_n).reshape(1, 16)
    off_n = (beta_n - mu_n * gamma_n / sig_n).reshape(1, 16)

    def norm_body(yr, ir, orr, o_ref):
        o_ref[...] = yr[...] * ir[...] + orr[...]

    return pl.pallas_call(
        norm_body,
        grid=(Nn // _BLKN,),
        in_specs=[
            pl.BlockSpec((_BLKN, 16), lambda i: (i, 0)),
            pl.BlockSpec((1, 16), lambda i: (0, 0)),
            pl.BlockSpec((1, 16), lambda i: (0, 0)),
        ],
        out_specs=pl.BlockSpec((_BLKN, 16), lambda i: (i, 0)),
        out_shape=jax.ShapeDtypeStruct((Nn, 16), _F32),
    )(y_raw, inv_n, off_n)


def kernel(coords, normals, curvatures, edge_index, W1, b1, W2, b2, W3, b3,
           Wp, bp, gamma_e, beta_e, gamma_n, beta_n):
    Nn = coords.shape[0]
    E = edge_index.shape[1]
    node_tab = jnp.concatenate([coords, normals, curvatures[:, :2]], axis=1)
    row2d = edge_index[0].reshape(E // _BS, _BS)
    col2d = edge_index[1].reshape(E // _BS, _BS)

    row_feat, col_feat = _stage1_gather(node_tab, row2d, col2d)
    h2, estats = _stage2_mlp(row_feat.T, col_feat.T, W1, b1, W2, b2, W3, b3)
    acc2, deg2 = _stage3_scatter(h2, row2d, col2d, Nn)
    return _stage4_node(acc2, deg2.T, estats, E, Wp, bp, gamma_e, beta_e,
                        gamma_n, beta_n)
